# Initial kernel scaffold; baseline (speedup 1.0000x reference)
import functools

import jax
import jax.numpy as jnp
from jax import lax
from jax.experimental import pallas as pl
from jax.experimental.pallas import tpu as pltpu
from jax.experimental.pallas import tpu_sc as plsc

N_CORES = 2
N_SUBCORES = 16
NW = N_CORES * N_SUBCORES
LANES = 16

B = 4096
H = 200
D = 32
NL = 10
SPW = B // NW
C0, C1 = 128, 72


def _pool_body(x_hbm, table_hbm, out_hbm, idx_v, rows_v, pooled_v, sem):
    wid = lax.axis_index("s") * N_CORES + lax.axis_index("c")
    base = wid * (SPW * H)
    pltpu.sync_copy(x_hbm.at[pl.ds(base, SPW * H)], idx_v)

    @pl.loop(0, SPW)
    def _(s):
        off = s * H
        pltpu.async_copy(
            table_hbm.at[idx_v.at[pl.ds(off, C0)]],
            rows_v.at[pl.ds(0, C0)], sem).wait()
        pltpu.async_copy(
            table_hbm.at[idx_v.at[pl.ds(off + C0, C1)]],
            rows_v.at[pl.ds(C0, C1)], sem).wait()

        def red(j, carry):
            a0, a1 = carry
            a0 = jnp.maximum(a0, rows_v[j, pl.ds(0, LANES)])
            a1 = jnp.maximum(a1, rows_v[j, pl.ds(LANES, LANES)])
            return a0, a1

        a0, a1 = lax.fori_loop(
            1, H, red,
            (rows_v[0, pl.ds(0, LANES)], rows_v[0, pl.ds(LANES, LANES)]))
        pooled_v[s, pl.ds(0, LANES)] = a0
        pooled_v[s, pl.ds(LANES, LANES)] = a1

    pltpu.sync_copy(pooled_v, out_hbm.at[pl.ds(wid * SPW, SPW)])


def _mm_body(p_ref, w_ref, b_ref, o_ref):
    o_ref[...] = lax.dot_general(
        p_ref[...], w_ref[...], (((1,), (1,)), ((), ())),
        preferred_element_type=jnp.float32) + b_ref[...]


def kernel(x, table, W, b):
    x_flat = x.reshape(-1).astype(jnp.int32)

    pool_call = pl.kernel(
        _pool_body,
        out_type=jax.ShapeDtypeStruct((B, D), jnp.float32),
        mesh=plsc.VectorSubcoreMesh(core_axis_name="c", subcore_axis_name="s"),
        scratch_types=[
            pltpu.VMEM((SPW * H,), jnp.int32),
            pltpu.VMEM((H, D), jnp.float32),
            pltpu.VMEM((SPW, D), jnp.float32),
            pltpu.SemaphoreType.DMA,
        ],
    )
    pooled = pool_call(x_flat, table)

    logits = pl.pallas_call(
        _mm_body,
        out_shape=jax.ShapeDtypeStruct((B, NL), jnp.float32),
    )(pooled, W, b.reshape(1, NL))
    return logits


# speedup vs baseline: 25.9341x; 25.9341x over previous
import functools

import jax
import jax.numpy as jnp
from jax import lax
from jax.experimental import pallas as pl
from jax.experimental.pallas import tpu as pltpu
from jax.experimental.pallas import tpu_sc as plsc

N_CORES = 2
N_SUBCORES = 16
NW = N_CORES * N_SUBCORES
LANES = 16

B = 4096
H = 200
D = 32
NL = 10
SPW = B // NW
C0, C1 = 128, 72


def _pool_body(x_hbm, table_hbm, out_hbm, idx_v, rows_v, pooled_v, sem):
    wid = lax.axis_index("s") * N_CORES + lax.axis_index("c")
    base = wid * (SPW * H)
    pltpu.sync_copy(x_hbm.at[pl.ds(base, SPW * H)], idx_v)

    @pl.loop(0, SPW)
    def _(s):
        off = s * H
        pltpu.async_copy(
            table_hbm.at[idx_v.at[pl.ds(off, C0)]],
            rows_v.at[pl.ds(0, C0)], sem).wait()
        pltpu.async_copy(
            table_hbm.at[idx_v.at[pl.ds(off + C0, C1)]],
            rows_v.at[pl.ds(C0, C1)], sem).wait()

        def red(j, carry):
            a0, a1 = carry
            a0 = jnp.maximum(a0, rows_v[j, pl.ds(0, LANES)])
            a1 = jnp.maximum(a1, rows_v[j, pl.ds(LANES, LANES)])
            return a0, a1

        a0, a1 = lax.fori_loop(
            1, H, red,
            (rows_v[0, pl.ds(0, LANES)], rows_v[0, pl.ds(LANES, LANES)]))
        pooled_v[s, pl.ds(0, LANES)] = a0
        pooled_v[s, pl.ds(LANES, LANES)] = a1

    pltpu.sync_copy(pooled_v, out_hbm.at[pl.ds(wid * SPW, SPW)])


def _mm_body(p_ref, w_ref, b_ref, o_ref):
    o_ref[...] = lax.dot_general(
        p_ref[...], w_ref[...], (((1,), (1,)), ((), ())),
        preferred_element_type=jnp.float32) + b_ref[...]


def kernel(x, table, W, b):
    x_flat = x.reshape(-1).astype(jnp.int32)

    pool_call = pl.kernel(
        _pool_body,
        out_type=jax.ShapeDtypeStruct((B, D), jnp.float32),
        mesh=plsc.VectorSubcoreMesh(core_axis_name="c", subcore_axis_name="s"),
        compiler_params=pltpu.CompilerParams(use_tc_tiling_on_sc=False),
        scratch_types=[
            pltpu.VMEM((SPW * H,), jnp.int32),
            pltpu.VMEM((H, D), jnp.float32),
            pltpu.VMEM((SPW, D), jnp.float32),
            pltpu.SemaphoreType.DMA,
        ],
    )
    pooled = pool_call(x_flat, table)

    logits = pl.pallas_call(
        _mm_body,
        out_shape=jax.ShapeDtypeStruct((B, NL), jnp.float32),
    )(pooled, W, b.reshape(1, NL))
    return logits


# speedup vs baseline: 34.6322x; 1.3354x over previous
import functools

import jax
import jax.numpy as jnp
from jax import lax
from jax.experimental import pallas as pl
from jax.experimental.pallas import tpu as pltpu
from jax.experimental.pallas import tpu_sc as plsc

N_CORES = 2
N_SUBCORES = 16
NW = N_CORES * N_SUBCORES
LANES = 16

B = 4096
H = 200
D = 32
NL = 10
SPW = B // NW
C0, C1 = 128, 72


UNROLL = 8


def _pool_body(x_hbm, table_hbm, out_hbm, idx_v, rows_a, rows_b, pooled_v,
               sem_a, sem_b):
    wid = lax.axis_index("s") * N_CORES + lax.axis_index("c")
    base = wid * (SPW * H)
    pltpu.sync_copy(x_hbm.at[pl.ds(base, SPW * H)], idx_v)

    def issue(s, rows_v, sem):
        off = s * H
        pltpu.async_copy(
            table_hbm.at[idx_v.at[pl.ds(off, C0)]],
            rows_v.at[pl.ds(0, C0)], sem)
        pltpu.async_copy(
            table_hbm.at[idx_v.at[pl.ds(off + C0, C1)]],
            rows_v.at[pl.ds(C0, C1)], sem)

    def drain(s, rows_v, sem):
        off = s * H
        pltpu.make_async_copy(
            table_hbm.at[idx_v.at[pl.ds(off, C0)]],
            rows_v.at[pl.ds(0, C0)], sem).wait()
        pltpu.make_async_copy(
            table_hbm.at[idx_v.at[pl.ds(off + C0, C1)]],
            rows_v.at[pl.ds(C0, C1)], sem).wait()

    def reduce(s, rows_v):
        def red(i, carry):
            a0, a1 = carry
            for u in range(UNROLL):
                j = i * UNROLL + u
                a0 = jnp.maximum(a0, rows_v[j, pl.ds(0, LANES)])
                a1 = jnp.maximum(a1, rows_v[j, pl.ds(LANES, LANES)])
            return a0, a1

        ninf = jnp.full((LANES,), -jnp.inf, jnp.float32)
        a0, a1 = lax.fori_loop(0, H // UNROLL, red, (ninf, ninf))
        pooled_v[s, pl.ds(0, LANES)] = a0
        pooled_v[s, pl.ds(LANES, LANES)] = a1

    issue(0, rows_a, sem_a)

    @pl.loop(0, SPW, step=2)
    def _(s):
        issue(s + 1, rows_b, sem_b)
        drain(s, rows_a, sem_a)
        reduce(s, rows_a)

        @pl.when(s + 2 < SPW)
        def _():
            issue(s + 2, rows_a, sem_a)

        drain(s + 1, rows_b, sem_b)
        reduce(s + 1, rows_b)

    pltpu.sync_copy(pooled_v, out_hbm.at[pl.ds(wid * SPW, SPW)])


def _mm_body(p_ref, w_ref, b_ref, o_ref):
    o_ref[...] = lax.dot_general(
        p_ref[...], w_ref[...], (((1,), (1,)), ((), ())),
        preferred_element_type=jnp.float32) + b_ref[...]


def kernel(x, table, W, b):
    x_flat = x.reshape(-1).astype(jnp.int32)

    pool_call = pl.kernel(
        _pool_body,
        out_type=jax.ShapeDtypeStruct((B, D), jnp.float32),
        mesh=plsc.VectorSubcoreMesh(core_axis_name="c", subcore_axis_name="s"),
        compiler_params=pltpu.CompilerParams(use_tc_tiling_on_sc=False),
        scratch_types=[
            pltpu.VMEM((SPW * H,), jnp.int32),
            pltpu.VMEM((H, D), jnp.float32),
            pltpu.VMEM((H, D), jnp.float32),
            pltpu.VMEM((SPW, D), jnp.float32),
            pltpu.SemaphoreType.DMA,
            pltpu.SemaphoreType.DMA,
        ],
    )
    pooled = pool_call(x_flat, table)

    logits = pl.pallas_call(
        _mm_body,
        out_shape=jax.ShapeDtypeStruct((B, NL), jnp.float32),
    )(pooled, W, b.reshape(1, NL))
    return logits


# speedup vs baseline: 52.4240x; 1.5137x over previous
import dataclasses
import functools

import jax
import jax.numpy as jnp
from jax import lax
from jax.experimental import pallas as pl
from jax.experimental.pallas import tpu as pltpu
from jax.experimental.pallas import tpu_sc as plsc

N_CORES = 2
N_SUBCORES = 16
NW = N_CORES * N_SUBCORES
LANES = 16

N_EMB = 1000000
B = 4096
H = 200
D = 32
NL = 10
SPW = B // NW
C0, C1 = 128, 72


UNROLL = 8

_SC_PARAMS = pltpu.CompilerParams(use_tc_tiling_on_sc=False)
if "needs_layout_passes" in pltpu.CompilerParams.__dataclass_fields__:
    _SC_PARAMS = dataclasses.replace(_SC_PARAMS, needs_layout_passes=False)


def _pool_body(x_hbm, table_hbm, out_hbm, idx_v, rows_a, rows_b, pooled_v,
               sem_a, sem_b):
    wid = lax.axis_index("s") * N_CORES + lax.axis_index("c")
    base = wid * (SPW * H)
    pltpu.sync_copy(x_hbm.at[pl.ds(base, SPW * H)], idx_v)

    @pl.loop(0, SPW * H, step=LANES)
    def _(i):
        v = idx_v[pl.ds(i, LANES)]
        u = jnp.bitwise_and(v, 511)
        idx_v[pl.ds(i, LANES)] = (
            (v - u)
            + jnp.left_shift(jnp.bitwise_and(u, 127), 2)
            + lax.shift_right_logical(u, 7))

    def issue(s, rows_v, sem):
        off = s * H
        pltpu.async_copy(
            table_hbm.at[idx_v.at[pl.ds(off, C0)]],
            rows_v.at[pl.ds(0, C0)], sem)
        pltpu.async_copy(
            table_hbm.at[idx_v.at[pl.ds(off + C0, C1)]],
            rows_v.at[pl.ds(C0, C1)], sem)

    def drain(s, rows_v, sem):
        off = s * H
        pltpu.make_async_copy(
            table_hbm.at[idx_v.at[pl.ds(off, C0)]],
            rows_v.at[pl.ds(0, C0)], sem).wait()
        pltpu.make_async_copy(
            table_hbm.at[idx_v.at[pl.ds(off + C0, C1)]],
            rows_v.at[pl.ds(C0, C1)], sem).wait()

    def reduce(s, rows_v):
        def red(i, carry):
            a0, a1 = carry
            for u in range(UNROLL):
                j = i * UNROLL + u
                a0 = jnp.maximum(a0, rows_v[j, pl.ds(0, LANES)])
                a1 = jnp.maximum(a1, rows_v[j, pl.ds(LANES, LANES)])
            return a0, a1

        ninf = jnp.full((LANES,), -jnp.inf, jnp.float32)
        a0, a1 = lax.fori_loop(0, H // UNROLL, red, (ninf, ninf))
        pooled_v[s, pl.ds(0, LANES)] = a0
        pooled_v[s, pl.ds(LANES, LANES)] = a1

    issue(0, rows_a, sem_a)

    @pl.loop(0, SPW, step=2)
    def _(s):
        issue(s + 1, rows_b, sem_b)
        drain(s, rows_a, sem_a)
        reduce(s, rows_a)

        @pl.when(s + 2 < SPW)
        def _():
            issue(s + 2, rows_a, sem_a)

        drain(s + 1, rows_b, sem_b)
        reduce(s + 1, rows_b)

    pltpu.sync_copy(pooled_v, out_hbm.at[pl.ds(wid * SPW, SPW)])


RC = 2560
NSTEP = (1000000 + RC - 1) // RC
N_PAD = NSTEP * RC


def _relayout_body(xt_ref, o_ref):
    x = xt_ref[...]
    for g in range(RC // 512):
        y = jnp.concatenate(
            [x[:, 512 * g + 128 * a:512 * g + 128 * (a + 1)] for a in range(4)],
            axis=0)
        o_ref[pl.ds(128 * g, 128), :] = y.T


def _mm_body(p_ref, w_ref, b_ref, o_ref):
    o_ref[...] = lax.dot_general(
        p_ref[...], w_ref[...], (((1,), (1,)), ((), ())),
        preferred_element_type=jnp.float32) + b_ref[...]


def kernel(x, table, W, b):
    x_flat = x.reshape(-1).astype(jnp.int32)

    pool_call = pl.kernel(
        _pool_body,
        out_type=jax.ShapeDtypeStruct((B, D), jnp.float32),
        mesh=plsc.VectorSubcoreMesh(core_axis_name="c", subcore_axis_name="s"),
        compiler_params=_SC_PARAMS,
        scratch_types=[
            pltpu.VMEM((SPW * H,), jnp.int32),
            pltpu.VMEM((H, D), jnp.float32),
            pltpu.VMEM((H, D), jnp.float32),
            pltpu.VMEM((SPW, D), jnp.float32),
            pltpu.SemaphoreType.DMA,
            pltpu.SemaphoreType.DMA,
        ],
    )
    t128 = pl.pallas_call(
        _relayout_body,
        grid=(NSTEP,),
        in_specs=[pl.BlockSpec((D, RC), lambda i: (0, i))],
        out_specs=pl.BlockSpec((RC * D // 128, 128), lambda i: (i, 0)),
        out_shape=jax.ShapeDtypeStruct((N_PAD * D // 128, 128), jnp.float32),
    )(table.T)
    pooled = pool_call(x_flat, t128.reshape(N_PAD, D))

    logits = pl.pallas_call(
        _mm_body,
        out_shape=jax.ShapeDtypeStruct((B, NL), jnp.float32),
    )(pooled, W, b.reshape(1, NL))
    return logits


# speedup vs baseline: 70.5447x; 1.3457x over previous
import dataclasses
import functools

import jax
import jax.numpy as jnp
from jax import lax
from jax.experimental import pallas as pl
from jax.experimental.pallas import tpu as pltpu
from jax.experimental.pallas import tpu_sc as plsc

N_CORES = 2
N_SUBCORES = 16
NW = N_CORES * N_SUBCORES
LANES = 16

N_EMB = 1000000
B = 4096
H = 200
D = 32
NL = 10
SPW = B // NW
C0, C1 = 128, 72


UNROLL = 8

_SC_PARAMS = pltpu.CompilerParams(use_tc_tiling_on_sc=False)
if "needs_layout_passes" in pltpu.CompilerParams.__dataclass_fields__:
    _SC_PARAMS = dataclasses.replace(_SC_PARAMS, needs_layout_passes=False)


def _pool_body(x_hbm, table_hbm, out_hbm, idx_v, rows_a, rows_b, pooled_v,
               sem_a, sem_b):
    wid = lax.axis_index("s") * N_CORES + lax.axis_index("c")
    base = wid * (SPW * H)
    pltpu.sync_copy(x_hbm.at[pl.ds(base, SPW * H)], idx_v)

    @pl.loop(0, SPW * H, step=LANES)
    def _(i):
        v = idx_v[pl.ds(i, LANES)]
        u = jnp.bitwise_and(v, 511)
        idx_v[pl.ds(i, LANES)] = (
            (v - u)
            + jnp.left_shift(jnp.bitwise_and(u, 127), 2)
            + lax.shift_right_logical(u, 7))

    def issue(s, rows_v, sem):
        off = s * H
        pltpu.async_copy(
            table_hbm.at[idx_v.at[pl.ds(off, C0)]],
            rows_v.at[pl.ds(0, C0)], sem)
        pltpu.async_copy(
            table_hbm.at[idx_v.at[pl.ds(off + C0, C1)]],
            rows_v.at[pl.ds(C0, C1)], sem)

    def drain(s, rows_v, sem):
        off = s * H
        pltpu.make_async_copy(
            table_hbm.at[idx_v.at[pl.ds(off, C0)]],
            rows_v.at[pl.ds(0, C0)], sem).wait()
        pltpu.make_async_copy(
            table_hbm.at[idx_v.at[pl.ds(off + C0, C1)]],
            rows_v.at[pl.ds(C0, C1)], sem).wait()

    def reduce(s, rows_v):
        def red(i, carry):
            a0, a1 = carry
            for u in range(UNROLL):
                j = i * UNROLL + u
                a0 = jnp.maximum(a0, rows_v[j, pl.ds(0, LANES)])
                a1 = jnp.maximum(a1, rows_v[j, pl.ds(LANES, LANES)])
            return a0, a1

        ninf = jnp.full((LANES,), -jnp.inf, jnp.float32)
        a0, a1 = lax.fori_loop(0, H // UNROLL, red, (ninf, ninf))
        pooled_v[s, pl.ds(0, LANES)] = a0
        pooled_v[s, pl.ds(LANES, LANES)] = a1

    issue(0, rows_a, sem_a)

    @pl.loop(0, SPW, step=2)
    def _(s):
        issue(s + 1, rows_b, sem_b)
        drain(s, rows_a, sem_a)
        reduce(s, rows_a)

        @pl.when(s + 2 < SPW)
        def _():
            issue(s + 2, rows_a, sem_a)

        drain(s + 1, rows_b, sem_b)
        reduce(s + 1, rows_b)

    pltpu.sync_copy(pooled_v, out_hbm.at[pl.ds(wid * SPW, SPW)])


RC = 5120
NSTEP = (1000000 + RC - 1) // RC
N_PAD = NSTEP * RC


def _relayout_body(xt_ref, o_ref):
    x = xt_ref[...]
    for g in range(RC // 512):
        y = jnp.concatenate(
            [x[:, 512 * g + 128 * a:512 * g + 128 * (a + 1)] for a in range(4)],
            axis=0)
        o_ref[pl.ds(128 * g, 128), :] = y.T


def _mm_body(p_ref, w_ref, b_ref, o_ref):
    o_ref[...] = lax.dot_general(
        p_ref[...], w_ref[...], (((1,), (1,)), ((), ())),
        preferred_element_type=jnp.float32) + b_ref[...]


def kernel(x, table, W, b):
    x_flat = x.reshape(-1).astype(jnp.int32)

    pool_call = pl.kernel(
        _pool_body,
        out_type=jax.ShapeDtypeStruct((B, D), jnp.float32),
        mesh=plsc.VectorSubcoreMesh(core_axis_name="c", subcore_axis_name="s"),
        compiler_params=_SC_PARAMS,
        scratch_types=[
            pltpu.VMEM((SPW * H,), jnp.int32),
            pltpu.VMEM((H, D), jnp.float32),
            pltpu.VMEM((H, D), jnp.float32),
            pltpu.VMEM((SPW, D), jnp.float32),
            pltpu.SemaphoreType.DMA,
            pltpu.SemaphoreType.DMA,
        ],
    )
    t128 = pl.pallas_call(
        _relayout_body,
        grid=(NSTEP,),
        in_specs=[pl.BlockSpec((D, RC), lambda i: (0, i))],
        out_specs=pl.BlockSpec((RC * D // 128, 128), lambda i: (i, 0)),
        out_shape=jax.ShapeDtypeStruct((N_PAD * D // 128, 128), jnp.float32),
        compiler_params=pltpu.CompilerParams(
            dimension_semantics=("parallel",)),
    )(table.T)
    pooled = pool_call(x_flat, t128.reshape(N_PAD, D))

    logits = pl.pallas_call(
        _mm_body,
        out_shape=jax.ShapeDtypeStruct((B, NL), jnp.float32),
    )(pooled, W, b.reshape(1, NL))
    return logits
